# baseline (speedup 1.0000x reference)
import functools

import jax
import jax.numpy as jnp
from jax import lax
from jax.experimental import pallas as pl
from jax.experimental.pallas import tpu as pltpu
from jax.experimental.pallas import tpu_sc as plsc

_HIDDEN = 1024
_NC = 2
_NS = 16
_NW = _NC * _NS
_L = 16
_NPR = 63


@functools.cache
def _sc_lookup(total: int, n_rows: int):
    per_w = total // _NW
    pairs_w = per_w // 2
    mesh = plsc.VectorSubcoreMesh(core_axis_name="c", subcore_axis_name="s")

    @functools.partial(
        pl.kernel,
        out_type=jax.ShapeDtypeStruct((total // 2, 2, _HIDDEN), jnp.float32),
        mesh=mesh,
        compiler_params=pltpu.CompilerParams(
            use_tc_tiling_on_sc=False, needs_layout_passes=False
        ),
        scratch_types=[
            pltpu.VMEM((per_w,), jnp.int32),
            pltpu.VMEM((pairs_w,), jnp.int32),
            pltpu.VMEM((2 * _NPR, _HIDDEN), jnp.float32),
            pltpu.SemaphoreType.DMA,
            pltpu.SemaphoreType.DMA,
        ],
    )
    def k(ids_hbm, pair_hbm, out_hbm, idx_v, pid_v, pair_v, tsem, rsem):
        wid = lax.axis_index("s") * _NC + lax.axis_index("c")
        base = wid * pairs_w
        cp_t = pltpu.async_copy(pair_hbm, pair_v, tsem)
        pltpu.sync_copy(ids_hbm.at[wid], idx_v)
        iota = lax.iota(jnp.int32, _L)

        def pid_blk(g, _):
            ev = plsc.load_gather(idx_v, [iota * 2 + g * (2 * _L)])
            od = plsc.load_gather(idx_v, [iota * 2 + g * (2 * _L) + 1])
            pid_v[pl.ds(g * _L, _L)] = ev * n_rows + od
            return ()

        lax.fori_loop(0, pairs_w // _L, pid_blk, (), unroll=False)
        cp_t.wait()

        def fire(p, _):
            vec = pid_v[pl.ds((p // _L) * _L, _L)]
            pid = jnp.max(jnp.where(iota == p % _L, vec, 0))
            roff = jnp.where(pid == n_rows * n_rows - 1, 111, 2 * pid)
            pltpu.async_copy(
                pair_v.at[pl.ds(roff, 2)], out_hbm.at[base + p], rsem
            )
            return ()

        lax.fori_loop(0, pairs_w, fire, (), unroll=False)

        def drain(p, _):
            pltpu.make_async_copy(
                pair_v.at[pl.ds(0, 2)], out_hbm.at[base], rsem
            ).wait()
            return ()

        lax.fori_loop(0, pairs_w, drain, (), unroll=False)

    return k


def kernel(ids, indicator_embs):
    b, s = ids.shape
    total = b * s
    n_rows = indicator_embs.shape[0]
    ids_w = ids.astype(jnp.int32).reshape(_NW, total // _NW)
    codes = jnp.arange(_NPR, dtype=jnp.int32)
    idx126 = jnp.stack([codes // n_rows, codes % n_rows], axis=1).reshape(-1)
    pair_hbm = indicator_embs[idx126]
    out = _sc_lookup(total, n_rows)(ids_w, pair_hbm)
    return out.reshape(b, s, _HIDDEN)
